# Initial kernel scaffold; baseline (speedup 1.0000x reference)
import jax
import jax.numpy as jnp
from jax.experimental import pallas as pl


def kernel(node_features, hyperedge_index, W, b):
    raise NotImplementedError("write your pallas kernel here")


# speedup vs baseline: 3.5133x; 3.5133x over previous
import functools

import jax
import jax.numpy as jnp
from jax import lax
from jax.experimental import pallas as pl
from jax.experimental.pallas import tpu as pltpu
from jax.experimental.pallas import tpu_sc as plsc

NUM_HYPEREDGES = 2500

_IB = 128
_K = 5
_ZR = 160


def _pad_len(n, mult):
    return ((n + mult - 1) // mult) * mult


def _make_sc_segsum(e_pad, out_rows, nc, ns, fh,
                    seg_rows=0, g_rows=0, with_degs=False):
    blk = _K * _IB
    ppt = e_pad // ns
    chunks = ppt // blk
    gr_pc = e_pad // _IB

    mesh = plsc.VectorSubcoreMesh(core_axis_name="c", subcore_axis_name="s")

    o_stripe = out_rows // 16
    seg_stripe = seg_rows // 16 if with_degs else 0
    g_stripe = g_rows // 16 if with_degs else 0

    def body(table_hbm, gidx2_hbm, sidx2_hbm, graw2_hbm, z2d_hbm, z1d_hbm,
             ones_hbm, *refs):
        if with_degs:
            (acc_out, segdeg_out, gdeg_out,
             acc_sh, segdeg_sh, gdeg_sh,
             gidx_v, sidx_v, graw_v, rows_v, stage_v, stage1_v, ones_v,
             sem) = refs
        else:
            (acc_out, acc_sh,
             gidx_v, sidx_v, rows_v, stage_v, sem) = refs
        c = lax.axis_index("c")
        s = lax.axis_index("s")
        pltpu.sync_copy(z2d_hbm.at[pl.ds(0, _ZR)], stage_v)
        for k in range(o_stripe // _ZR):
            pltpu.sync_copy(stage_v,
                            acc_sh.at[pl.ds(s * o_stripe + k * _ZR, _ZR)])
        if with_degs:
            pltpu.sync_copy(z1d_hbm.at[pl.ds(0, max(seg_stripe, g_stripe))],
                            stage1_v)
            pltpu.sync_copy(stage1_v.at[pl.ds(0, seg_stripe)],
                            segdeg_sh.at[pl.ds(s * seg_stripe, seg_stripe)])
            pltpu.sync_copy(stage1_v.at[pl.ds(0, g_stripe)],
                            gdeg_sh.at[pl.ds(s * g_stripe, g_stripe)])
            pltpu.sync_copy(ones_hbm, ones_v)
        plsc.subcore_barrier()

        def chunk(g, carry):
            row0 = s * (ppt // _IB) + g * _K
            pltpu.sync_copy(gidx2_hbm.at[pl.ds(c * gr_pc + row0, _K)],
                            gidx_v)
            pltpu.sync_copy(sidx2_hbm.at[pl.ds(row0, _K)], sidx_v)
            if with_degs:
                pltpu.sync_copy(graw2_hbm.at[pl.ds(row0, _K)], graw_v)
            for j in range(_K):
                pltpu.async_copy(table_hbm.at[gidx_v.at[j]],
                                 rows_v.at[pl.ds(j * _IB, _IB)], sem).wait()
                pltpu.sync_copy(rows_v.at[pl.ds(j * _IB, _IB)],
                                acc_sh.at[sidx_v.at[j]], add=True)
                if with_degs:
                    pltpu.sync_copy(ones_v,
                                    segdeg_sh.at[sidx_v.at[j]], add=True)
                    pltpu.sync_copy(ones_v,
                                    gdeg_sh.at[graw_v.at[j]], add=True)
            return carry

        lax.fori_loop(0, chunks, chunk, 0)
        plsc.subcore_barrier()
        for k in range(o_stripe // _ZR):
            pltpu.sync_copy(acc_sh.at[pl.ds(s * o_stripe + k * _ZR, _ZR)],
                            stage_v)
            pltpu.sync_copy(stage_v,
                            acc_out.at[c, pl.ds(s * o_stripe + k * _ZR,
                                                _ZR)])
        if with_degs:
            pltpu.sync_copy(
                segdeg_sh.at[pl.ds(s * seg_stripe, seg_stripe)],
                stage1_v.at[pl.ds(0, seg_stripe)])
            pltpu.sync_copy(
                stage1_v.at[pl.ds(0, seg_stripe)],
                segdeg_out.at[c, pl.ds(s * seg_stripe, seg_stripe)])
            pltpu.sync_copy(gdeg_sh.at[pl.ds(s * g_stripe, g_stripe)],
                            stage1_v.at[pl.ds(0, g_stripe)])
            pltpu.sync_copy(
                stage1_v.at[pl.ds(0, g_stripe)],
                gdeg_out.at[c, pl.ds(s * g_stripe, g_stripe)])

    out_type = [jax.ShapeDtypeStruct((nc, out_rows, fh), jnp.float32)]
    scratch = [pltpu.VMEM_SHARED((out_rows, fh), jnp.float32)]
    if with_degs:
        out_type += [
            jax.ShapeDtypeStruct((nc, seg_rows, 16), jnp.float32),
            jax.ShapeDtypeStruct((nc, g_rows, 16), jnp.float32)]
        scratch += [pltpu.VMEM_SHARED((seg_rows, 16), jnp.float32),
                    pltpu.VMEM_SHARED((g_rows, 16), jnp.float32)]
    scratch += [pltpu.VMEM((_K, _IB), jnp.int32),
                pltpu.VMEM((_K, _IB), jnp.int32)]
    if with_degs:
        scratch += [pltpu.VMEM((_K, _IB), jnp.int32)]
    scratch += [pltpu.VMEM((_K * _IB, fh), jnp.float32),
                pltpu.VMEM((_ZR, fh), jnp.float32)]
    if with_degs:
        scratch += [
            pltpu.VMEM((max(seg_rows, g_rows) // 16, 16), jnp.float32),
            pltpu.VMEM((_IB, 16), jnp.float32)]
    scratch += [pltpu.SemaphoreType.DMA]

    return pl.kernel(body, mesh=mesh, out_type=out_type,
                     scratch_types=scratch,
                     compiler_params=pltpu.CompilerParams(
                         use_tc_tiling_on_sc=False))


def _combine_norm_kernel(hp0_ref, hp1_ref, de_ref, out_ref):
    inv = (1.0 / jnp.maximum(de_ref[...], 1.0))
    out_ref[...] = jnp.concatenate([hp0_ref[...], hp1_ref[...]],
                                   axis=1) * inv


def _finish_kernel(op0_ref, op1_ref, dn_ref, w_ref, b_ref, out_ref):
    inv = 1.0 / jnp.maximum(dn_ref[...], 1.0)
    x = jnp.concatenate([op0_ref[...], op1_ref[...]], axis=1) * inv
    out_ref[...] = jnp.dot(x, w_ref[...],
                           preferred_element_type=jnp.float32) + b_ref[...]


def kernel(node_features, hyperedge_index, W, b):
    n, feat = node_features.shape
    e = hyperedge_index.shape[1]
    m = NUM_HYPEREDGES

    info = plsc.get_sparse_core_info()
    nc, ns = info.num_cores, info.num_subcores
    fh = feat // nc

    n_pad = _pad_len(n + 1, 16 * _ZR)
    m_pad = _pad_len(m + 1, 16 * _ZR)
    e_pad = _pad_len(e, ns * _K * _IB)

    node_idx = hyperedge_index[0]
    hedge_idx = hyperedge_index[1]
    if e_pad != e:
        node_idx = jnp.pad(node_idx, (0, e_pad - e), constant_values=n)
        hedge_idx = jnp.pad(hedge_idx, (0, e_pad - e), constant_values=m)
    x_pad = jnp.pad(node_features, ((0, n_pad - n), (0, 0)))
    x2 = x_pad.reshape(n_pad * nc, fh)

    core_off = jnp.arange(nc, dtype=jnp.int32)[:, None]
    nidx2 = (node_idx[None, :] * nc + core_off).reshape(-1, _IB)
    hidx2 = (hedge_idx[None, :] * nc + core_off).reshape(-1, _IB)
    nidx_2d = node_idx.reshape(-1, _IB)
    hidx_2d = hedge_idx.reshape(-1, _IB)

    z2d = jnp.zeros((_ZR, fh), jnp.float32)
    z1d = jnp.zeros((max(n_pad, m_pad) // 16 + 8, 16), jnp.float32)
    ones_c = jnp.ones((_IB, 16), jnp.float32)

    seg1 = _make_sc_segsum(e_pad, m_pad, nc, ns, fh,
                           seg_rows=m_pad, g_rows=n_pad, with_degs=True)
    hp, de3, dn3 = seg1(x2, nidx2, hidx_2d, nidx_2d, z2d, z1d, ones_c)
    de = de3[0, :, 0]
    dn = dn3[0, :, 0]

    hedge_feat = pl.pallas_call(
        _combine_norm_kernel,
        out_shape=jax.ShapeDtypeStruct((m_pad, feat), jnp.float32),
    )(hp[0], hp[1], de[:, None])
    hf2 = hedge_feat.reshape(m_pad * nc, fh)

    seg2 = _make_sc_segsum(e_pad, n_pad, nc, ns, fh)
    op = seg2(hf2, hidx2, nidx_2d, nidx_2d, z2d, z1d, ones_c)
    if isinstance(op, (list, tuple)):
        op = op[0]

    bn = 400 if n % 400 == 0 else n
    out = pl.pallas_call(
        _finish_kernel,
        grid=(n // bn,),
        in_specs=[
            pl.BlockSpec((bn, fh), lambda i: (i, 0)),
            pl.BlockSpec((bn, fh), lambda i: (i, 0)),
            pl.BlockSpec((bn, 1), lambda i: (i, 0)),
            pl.BlockSpec((feat, feat), lambda i: (0, 0)),
            pl.BlockSpec((1, feat), lambda i: (0, 0)),
        ],
        out_specs=pl.BlockSpec((bn, feat), lambda i: (i, 0)),
        out_shape=jax.ShapeDtypeStruct((n, feat), jnp.float32),
    )(op[0, :n], op[1, :n], dn[:n, None], W, b[None, :])
    return out


# speedup vs baseline: 4.0873x; 1.1634x over previous
import functools

import jax
import jax.numpy as jnp
from jax import lax
from jax.experimental import pallas as pl
from jax.experimental.pallas import tpu as pltpu
from jax.experimental.pallas import tpu_sc as plsc

NUM_HYPEREDGES = 2500

_IB = 128
_K = 5
_ZR = 80


def _pad_len(n, mult):
    return ((n + mult - 1) // mult) * mult


def _make_sc_segsum(e_pad, out_rows, nc, ns, fh,
                    seg_rows=0, g_rows=0, with_degs=False):
    blk = _K * _IB
    ppt = e_pad // ns
    chunks = ppt // blk
    gr_pc = e_pad // _IB

    mesh = plsc.VectorSubcoreMesh(core_axis_name="c", subcore_axis_name="s")

    o_stripe = out_rows // 16
    seg_stripe = seg_rows // 16 if with_degs else 0
    g_stripe = g_rows // 16 if with_degs else 0

    def body(table_hbm, gidx2_hbm, sidx2_hbm, graw2_hbm, z2d_hbm, z1d_hbm,
             ones_hbm, *refs):
        if with_degs:
            (acc_out, segdeg_out, gdeg_out,
             acc_sh, segdeg_sh, gdeg_sh,
             g0_v, g1_v, s0_v, s1_v, gr0_v, gr1_v, r0_v, r1_v,
             stage_v, stage1_v, ones_v,
             gsem0, gsem1, ssem0, ssem1) = refs
            grbuf = (gr0_v, gr1_v)
        else:
            (acc_out, acc_sh,
             g0_v, g1_v, s0_v, s1_v, r0_v, r1_v, stage_v,
             gsem0, gsem1, ssem0, ssem1) = refs
        gbuf = (g0_v, g1_v)
        sbuf = (s0_v, s1_v)
        rbuf = (r0_v, r1_v)
        gsem = (gsem0, gsem1)
        ssem = (ssem0, ssem1)
        c = lax.axis_index("c")
        s = lax.axis_index("s")
        pltpu.sync_copy(z2d_hbm.at[pl.ds(0, _ZR)], stage_v)
        for k in range(o_stripe // _ZR):
            pltpu.sync_copy(stage_v,
                            acc_sh.at[pl.ds(s * o_stripe + k * _ZR, _ZR)])
        if with_degs:
            pltpu.sync_copy(z1d_hbm.at[pl.ds(0, max(seg_stripe, g_stripe))],
                            stage1_v)
            pltpu.sync_copy(stage1_v.at[pl.ds(0, seg_stripe)],
                            segdeg_sh.at[pl.ds(s * seg_stripe, seg_stripe)])
            pltpu.sync_copy(stage1_v.at[pl.ds(0, g_stripe)],
                            gdeg_sh.at[pl.ds(s * g_stripe, g_stripe)])
            pltpu.sync_copy(ones_hbm, ones_v)
        plsc.subcore_barrier()

        def load_and_fire(g, b):
            row0 = s * (ppt // _IB) + g * _K
            pltpu.sync_copy(gidx2_hbm.at[pl.ds(c * gr_pc + row0, _K)],
                            gbuf[b])
            pltpu.sync_copy(sidx2_hbm.at[pl.ds(row0, _K)], sbuf[b])
            if with_degs:
                pltpu.sync_copy(graw2_hbm.at[pl.ds(row0, _K)], grbuf[b])
            for j in range(_K):
                pltpu.async_copy(table_hbm.at[gbuf[b].at[j]],
                                 rbuf[b].at[pl.ds(j * _IB, _IB)], gsem[b])

        def drain_gathers(b):
            for j in range(_K):
                pltpu.make_async_copy(
                    table_hbm.at[gbuf[b].at[j]],
                    rbuf[b].at[pl.ds(j * _IB, _IB)], gsem[b]).wait()

        def scatter(b):
            pend = []
            for j in range(_K):
                pend.append(pltpu.async_copy(
                    rbuf[b].at[pl.ds(j * _IB, _IB)],
                    acc_sh.at[sbuf[b].at[j]], ssem[b], add=True))
                if with_degs:
                    pend.append(pltpu.async_copy(
                        ones_v, segdeg_sh.at[sbuf[b].at[j]], ssem[b],
                        add=True))
                    pend.append(pltpu.async_copy(
                        ones_v, gdeg_sh.at[grbuf[b].at[j]], ssem[b],
                        add=True))
            for d in pend:
                d.wait()

        load_and_fire(0, 0)

        def pair(gg, carry):
            for b in range(2):
                g = 2 * gg + b
                gnext = jnp.minimum(g + 1, chunks - 1)
                load_and_fire(gnext, 1 - b)
                drain_gathers(b)
                scatter(b)
            return carry

        lax.fori_loop(0, chunks // 2, pair, 0)
        drain_gathers(0)
        plsc.subcore_barrier()
        for k in range(o_stripe // _ZR):
            pltpu.sync_copy(acc_sh.at[pl.ds(s * o_stripe + k * _ZR, _ZR)],
                            stage_v)
            pltpu.sync_copy(stage_v,
                            acc_out.at[c, pl.ds(s * o_stripe + k * _ZR,
                                                _ZR)])
        if with_degs:
            pltpu.sync_copy(
                segdeg_sh.at[pl.ds(s * seg_stripe, seg_stripe)],
                stage1_v.at[pl.ds(0, seg_stripe)])
            pltpu.sync_copy(
                stage1_v.at[pl.ds(0, seg_stripe)],
                segdeg_out.at[c, pl.ds(s * seg_stripe, seg_stripe)])
            pltpu.sync_copy(gdeg_sh.at[pl.ds(s * g_stripe, g_stripe)],
                            stage1_v.at[pl.ds(0, g_stripe)])
            pltpu.sync_copy(
                stage1_v.at[pl.ds(0, g_stripe)],
                gdeg_out.at[c, pl.ds(s * g_stripe, g_stripe)])

    out_type = [jax.ShapeDtypeStruct((nc, out_rows, fh), jnp.float32)]
    scratch = [pltpu.VMEM_SHARED((out_rows, fh), jnp.float32)]
    if with_degs:
        out_type += [
            jax.ShapeDtypeStruct((nc, seg_rows, 16), jnp.float32),
            jax.ShapeDtypeStruct((nc, g_rows, 16), jnp.float32)]
        scratch += [pltpu.VMEM_SHARED((seg_rows, 16), jnp.float32),
                    pltpu.VMEM_SHARED((g_rows, 16), jnp.float32)]
    scratch += [pltpu.VMEM((_K, _IB), jnp.int32),
                pltpu.VMEM((_K, _IB), jnp.int32),
                pltpu.VMEM((_K, _IB), jnp.int32),
                pltpu.VMEM((_K, _IB), jnp.int32)]
    if with_degs:
        scratch += [pltpu.VMEM((_K, _IB), jnp.int32),
                    pltpu.VMEM((_K, _IB), jnp.int32)]
    scratch += [pltpu.VMEM((_K * _IB, fh), jnp.float32),
                pltpu.VMEM((_K * _IB, fh), jnp.float32),
                pltpu.VMEM((_ZR, fh), jnp.float32)]
    if with_degs:
        scratch += [
            pltpu.VMEM((max(seg_rows, g_rows) // 16, 16), jnp.float32),
            pltpu.VMEM((_IB, 16), jnp.float32)]
    scratch += [pltpu.SemaphoreType.DMA, pltpu.SemaphoreType.DMA,
                pltpu.SemaphoreType.DMA, pltpu.SemaphoreType.DMA]

    return pl.kernel(body, mesh=mesh, out_type=out_type,
                     scratch_types=scratch,
                     compiler_params=pltpu.CompilerParams(
                         use_tc_tiling_on_sc=False))


def _combine_norm_kernel(hp0_ref, hp1_ref, de_ref, out_ref):
    inv = (1.0 / jnp.maximum(de_ref[...], 1.0))
    out_ref[...] = jnp.concatenate([hp0_ref[...], hp1_ref[...]],
                                   axis=1) * inv


def _finish_kernel(op0_ref, op1_ref, dn_ref, w_ref, b_ref, out_ref):
    inv = 1.0 / jnp.maximum(dn_ref[...], 1.0)
    x = jnp.concatenate([op0_ref[...], op1_ref[...]], axis=1) * inv
    out_ref[...] = jnp.dot(x, w_ref[...],
                           preferred_element_type=jnp.float32) + b_ref[...]


def kernel(node_features, hyperedge_index, W, b):
    n, feat = node_features.shape
    e = hyperedge_index.shape[1]
    m = NUM_HYPEREDGES

    info = plsc.get_sparse_core_info()
    nc, ns = info.num_cores, info.num_subcores
    fh = feat // nc

    n_pad = _pad_len(n + 1, 16 * _ZR)
    m_pad = _pad_len(m + 1, 16 * _ZR)
    e_pad = _pad_len(e, 2 * ns * _K * _IB)

    node_idx = hyperedge_index[0]
    hedge_idx = hyperedge_index[1]
    if e_pad != e:
        node_idx = jnp.pad(node_idx, (0, e_pad - e), constant_values=n)
        hedge_idx = jnp.pad(hedge_idx, (0, e_pad - e), constant_values=m)
    x_pad = jnp.pad(node_features, ((0, n_pad - n), (0, 0)))
    x2 = x_pad.reshape(n_pad * nc, fh)

    core_off = jnp.arange(nc, dtype=jnp.int32)[:, None]
    nidx2 = (node_idx[None, :] * nc + core_off).reshape(-1, _IB)
    hidx2 = (hedge_idx[None, :] * nc + core_off).reshape(-1, _IB)
    nidx_2d = node_idx.reshape(-1, _IB)
    hidx_2d = hedge_idx.reshape(-1, _IB)

    z2d = jnp.zeros((_ZR, fh), jnp.float32)
    z1d = jnp.zeros((max(n_pad, m_pad) // 16 + 8, 16), jnp.float32)
    ones_c = jnp.ones((_IB, 16), jnp.float32)

    seg1 = _make_sc_segsum(e_pad, m_pad, nc, ns, fh,
                           seg_rows=m_pad, g_rows=n_pad, with_degs=True)
    hp, de3, dn3 = seg1(x2, nidx2, hidx_2d, nidx_2d, z2d, z1d, ones_c)
    de = de3[0, :, 0]
    dn = dn3[0, :, 0]

    hedge_feat = pl.pallas_call(
        _combine_norm_kernel,
        out_shape=jax.ShapeDtypeStruct((m_pad, feat), jnp.float32),
    )(hp[0], hp[1], de[:, None])
    hf2 = hedge_feat.reshape(m_pad * nc, fh)

    seg2 = _make_sc_segsum(e_pad, n_pad, nc, ns, fh)
    op = seg2(hf2, hidx2, nidx_2d, nidx_2d, z2d, z1d, ones_c)
    if isinstance(op, (list, tuple)):
        op = op[0]

    bn = 400 if n % 400 == 0 else n
    out = pl.pallas_call(
        _finish_kernel,
        grid=(n // bn,),
        in_specs=[
            pl.BlockSpec((bn, fh), lambda i: (i, 0)),
            pl.BlockSpec((bn, fh), lambda i: (i, 0)),
            pl.BlockSpec((bn, 1), lambda i: (i, 0)),
            pl.BlockSpec((feat, feat), lambda i: (0, 0)),
            pl.BlockSpec((1, feat), lambda i: (0, 0)),
        ],
        out_specs=pl.BlockSpec((bn, feat), lambda i: (i, 0)),
        out_shape=jax.ShapeDtypeStruct((n, feat), jnp.float32),
    )(op[0, :n], op[1, :n], dn[:n, None], W, b[None, :])
    return out


# speedup vs baseline: 4.7933x; 1.1727x over previous
import functools

import jax
import jax.numpy as jnp
from jax import lax
from jax.experimental import pallas as pl
from jax.experimental.pallas import tpu as pltpu
from jax.experimental.pallas import tpu_sc as plsc

NUM_HYPEREDGES = 2500

_IB = 128
_K = 5
_ZR = 80


def _pad_len(n, mult):
    return ((n + mult - 1) // mult) * mult


def _make_sc_segsum(e_pad, out_rows, nc, ns, fh,
                    seg_rows=0, g_rows=0, with_degs=False,
                    table_rows=0):
    table_spmem = table_rows > 0
    blk = _K * _IB
    ppt = e_pad // ns
    chunks = ppt // blk

    mesh = plsc.VectorSubcoreMesh(core_axis_name="c", subcore_axis_name="s")

    o_stripe = out_rows // 16
    seg_stripe = seg_rows // 16 if with_degs else 0
    g_stripe = g_rows // 16 if with_degs else 0

    def body(table_hbm, gidx2_hbm, sidx2_hbm, graw2_hbm, z2d_hbm, z1d_hbm,
             ones_hbm, *refs):
        refs = list(refs)
        if with_degs:
            acc_out, segdeg_out, gdeg_out = refs[:3]
            del refs[:3]
        else:
            acc_out = refs.pop(0)
        acc_sh = refs.pop(0)
        if table_spmem:
            table_sh = refs.pop(0)
        if with_degs:
            segdeg_sh = refs.pop(0)
            gdeg_sh = refs.pop(0)
        g0_v, g1_v, s0_v, s1_v = refs[:4]
        del refs[:4]
        if with_degs:
            gr0_v, gr1_v = refs[:2]
            del refs[:2]
            grbuf = (gr0_v, gr1_v)
        r0_v, r1_v, stage_v = refs[:3]
        del refs[:3]
        if with_degs:
            stage1_v, ones_v = refs[:2]
            del refs[:2]
        gsem0, gsem1, ssem0, ssem1 = refs
        gbuf = (g0_v, g1_v)
        sbuf = (s0_v, s1_v)
        rbuf = (r0_v, r1_v)
        gsem = (gsem0, gsem1)
        ssem = (ssem0, ssem1)
        c = lax.axis_index("c")
        s = lax.axis_index("s")
        pltpu.sync_copy(z2d_hbm.at[pl.ds(0, _ZR)], stage_v)
        for k in range(o_stripe // _ZR):
            pltpu.sync_copy(stage_v,
                            acc_sh.at[pl.ds(s * o_stripe + k * _ZR, _ZR)])
        if with_degs:
            pltpu.sync_copy(z1d_hbm.at[pl.ds(0, max(seg_stripe, g_stripe))],
                            stage1_v)
            pltpu.sync_copy(stage1_v.at[pl.ds(0, seg_stripe)],
                            segdeg_sh.at[pl.ds(s * seg_stripe, seg_stripe)])
            pltpu.sync_copy(stage1_v.at[pl.ds(0, g_stripe)],
                            gdeg_sh.at[pl.ds(s * g_stripe, g_stripe)])
            pltpu.sync_copy(ones_hbm, ones_v)
        if table_spmem:
            t_stripe = table_rows // 16
            for k in range(t_stripe // _ZR):
                pltpu.sync_copy(
                    table_hbm.at[c, pl.ds(s * t_stripe + k * _ZR, _ZR)],
                    stage_v)
                pltpu.sync_copy(
                    stage_v,
                    table_sh.at[pl.ds(s * t_stripe + k * _ZR, _ZR)])
        plsc.subcore_barrier()
        gsrc = table_sh if table_spmem else table_hbm.at[c]

        def load_and_fire(g, b):
            row0 = s * (ppt // _IB) + g * _K
            pltpu.sync_copy(gidx2_hbm.at[pl.ds(row0, _K)], gbuf[b])
            pltpu.sync_copy(sidx2_hbm.at[pl.ds(row0, _K)], sbuf[b])
            if with_degs:
                pltpu.sync_copy(graw2_hbm.at[pl.ds(row0, _K)], grbuf[b])
            for j in range(_K):
                pltpu.async_copy(gsrc.at[gbuf[b].at[j]],
                                 rbuf[b].at[pl.ds(j * _IB, _IB)], gsem[b])

        def drain_gathers(b):
            for j in range(_K):
                pltpu.make_async_copy(
                    gsrc.at[gbuf[b].at[j]],
                    rbuf[b].at[pl.ds(j * _IB, _IB)], gsem[b]).wait()

        def scatter(b):
            pend = []
            for j in range(_K):
                pend.append(pltpu.async_copy(
                    rbuf[b].at[pl.ds(j * _IB, _IB)],
                    acc_sh.at[sbuf[b].at[j]], ssem[b], add=True))
                if with_degs:
                    pend.append(pltpu.async_copy(
                        ones_v, segdeg_sh.at[sbuf[b].at[j]], ssem[b],
                        add=True))
                    pend.append(pltpu.async_copy(
                        ones_v, gdeg_sh.at[grbuf[b].at[j]], ssem[b],
                        add=True))
            for d in pend:
                d.wait()

        load_and_fire(0, 0)

        def pair(gg, carry):
            for b in range(2):
                g = 2 * gg + b
                gnext = jnp.minimum(g + 1, chunks - 1)
                load_and_fire(gnext, 1 - b)
                drain_gathers(b)
                scatter(b)
            return carry

        lax.fori_loop(0, chunks // 2, pair, 0)
        drain_gathers(0)
        plsc.subcore_barrier()
        for k in range(o_stripe // _ZR):
            pltpu.sync_copy(
                acc_sh.at[pl.ds(s * o_stripe + k * _ZR, _ZR)], stage_v)
            pltpu.sync_copy(
                stage_v,
                acc_out.at[c, pl.ds(s * o_stripe + k * _ZR, _ZR)])
        if with_degs:
            pltpu.sync_copy(
                segdeg_sh.at[pl.ds(s * seg_stripe, seg_stripe)],
                stage1_v.at[pl.ds(0, seg_stripe)])
            pltpu.sync_copy(
                stage1_v.at[pl.ds(0, seg_stripe)],
                segdeg_out.at[c, pl.ds(s * seg_stripe, seg_stripe)])
            pltpu.sync_copy(gdeg_sh.at[pl.ds(s * g_stripe, g_stripe)],
                            stage1_v.at[pl.ds(0, g_stripe)])
            pltpu.sync_copy(
                stage1_v.at[pl.ds(0, g_stripe)],
                gdeg_out.at[c, pl.ds(s * g_stripe, g_stripe)])

    out_type = [jax.ShapeDtypeStruct((nc, out_rows, fh), jnp.float32)]
    scratch = [pltpu.VMEM_SHARED((out_rows, fh), jnp.float32)]
    if table_spmem:
        scratch += [pltpu.VMEM_SHARED((table_rows, fh), jnp.float32)]
    if with_degs:
        out_type += [
            jax.ShapeDtypeStruct((nc, seg_rows, 8), jnp.float32),
            jax.ShapeDtypeStruct((nc, g_rows, 8), jnp.float32)]
        scratch += [pltpu.VMEM_SHARED((seg_rows, 8), jnp.float32),
                    pltpu.VMEM_SHARED((g_rows, 8), jnp.float32)]
    scratch += [pltpu.VMEM((_K, _IB), jnp.int32),
                pltpu.VMEM((_K, _IB), jnp.int32),
                pltpu.VMEM((_K, _IB), jnp.int32),
                pltpu.VMEM((_K, _IB), jnp.int32)]
    if with_degs:
        scratch += [pltpu.VMEM((_K, _IB), jnp.int32),
                    pltpu.VMEM((_K, _IB), jnp.int32)]
    scratch += [pltpu.VMEM((_K * _IB, fh), jnp.float32),
                pltpu.VMEM((_K * _IB, fh), jnp.float32),
                pltpu.VMEM((_ZR, fh), jnp.float32)]
    if with_degs:
        scratch += [
            pltpu.VMEM((max(seg_rows, g_rows) // 16, 8), jnp.float32),
            pltpu.VMEM((_IB, 8), jnp.float32)]
    scratch += [pltpu.SemaphoreType.DMA, pltpu.SemaphoreType.DMA,
                pltpu.SemaphoreType.DMA, pltpu.SemaphoreType.DMA]

    return pl.kernel(body, mesh=mesh, out_type=out_type,
                     scratch_types=scratch,
                     compiler_params=pltpu.CompilerParams(
                         use_tc_tiling_on_sc=False))


def _combine_norm_kernel(hp_ref, de_ref, out_ref):
    inv = (1.0 / jnp.maximum(de_ref[...], 1.0))
    out_ref[...] = hp_ref[...] * inv[None]


def _finish_kernel(op0_ref, op1_ref, dn_ref, w_ref, b_ref, out_ref):
    inv = 1.0 / jnp.maximum(dn_ref[...], 1.0)
    x = jnp.concatenate([op0_ref[...], op1_ref[...]], axis=1) * inv
    out_ref[...] = jnp.dot(x, w_ref[...],
                           preferred_element_type=jnp.float32) + b_ref[...]


def kernel(node_features, hyperedge_index, W, b):
    n, feat = node_features.shape
    e = hyperedge_index.shape[1]
    m = NUM_HYPEREDGES

    info = plsc.get_sparse_core_info()
    nc, ns = info.num_cores, info.num_subcores
    fh = feat // nc

    n_pad = _pad_len(n + 1, 16 * _ZR)
    m_pad = _pad_len(m + 1, 16 * _ZR)
    e_pad = _pad_len(e, 2 * ns * _K * _IB)

    node_idx = hyperedge_index[0]
    hedge_idx = hyperedge_index[1]
    if e_pad != e:
        node_idx = jnp.pad(node_idx, (0, e_pad - e), constant_values=n)
        hedge_idx = jnp.pad(hedge_idx, (0, e_pad - e), constant_values=m)
    x_pad = jnp.pad(node_features, ((0, n_pad - n), (0, 0)))
    x_split = x_pad.reshape(n_pad, nc, fh).transpose(1, 0, 2)

    nidx_2d = node_idx.reshape(-1, _IB)
    hidx_2d = hedge_idx.reshape(-1, _IB)

    z2d = jnp.zeros((_ZR, fh), jnp.float32)
    z1d = jnp.zeros((max(n_pad, m_pad) // 16 + 8, 8), jnp.float32)
    ones_c = jnp.ones((_IB, 8), jnp.float32)

    seg1 = _make_sc_segsum(e_pad, m_pad, nc, ns, fh,
                           seg_rows=m_pad, g_rows=n_pad, with_degs=True)
    hp, de3, dn3 = seg1(x_split, nidx_2d, hidx_2d, nidx_2d, z2d, z1d,
                        ones_c)
    de = de3[0, :, 0]
    dn = dn3[0, :, 0]

    hf_split = pl.pallas_call(
        _combine_norm_kernel,
        out_shape=jax.ShapeDtypeStruct((nc, m_pad, fh), jnp.float32),
    )(hp, de[:, None])

    seg2 = _make_sc_segsum(e_pad, n_pad, nc, ns, fh)
    op = seg2(hf_split, hidx_2d, nidx_2d, nidx_2d[:1], z2d, z1d, ones_c)
    if isinstance(op, (list, tuple)):
        op = op[0]

    bn = 400 if n % 400 == 0 else n
    out = pl.pallas_call(
        _finish_kernel,
        grid=(n // bn,),
        in_specs=[
            pl.BlockSpec((bn, fh), lambda i: (i, 0)),
            pl.BlockSpec((bn, fh), lambda i: (i, 0)),
            pl.BlockSpec((bn, 1), lambda i: (i, 0)),
            pl.BlockSpec((feat, feat), lambda i: (0, 0)),
            pl.BlockSpec((1, feat), lambda i: (0, 0)),
        ],
        out_specs=pl.BlockSpec((bn, feat), lambda i: (i, 0)),
        out_shape=jax.ShapeDtypeStruct((n, feat), jnp.float32),
    )(op[0, :n], op[1, :n], dn[:n, None], W, b[None, :])
    return out


# speedup vs baseline: 5.4369x; 1.1343x over previous
import functools

import jax
import jax.numpy as jnp
from jax import lax
from jax.experimental import pallas as pl
from jax.experimental.pallas import tpu as pltpu
from jax.experimental.pallas import tpu_sc as plsc

NUM_HYPEREDGES = 2500

_IB = 128
_K = 5
_ZR = 80


def _pad_len(n, mult):
    return ((n + mult - 1) // mult) * mult


def _make_sc_segsum(e_pad, out_rows, nc, ns, fh,
                    seg_rows=0, g_rows=0, with_degs=False,
                    table_rows=0):
    table_spmem = table_rows > 0
    blk = _K * _IB
    ppt = e_pad // ns
    chunks = ppt // blk

    mesh = plsc.VectorSubcoreMesh(core_axis_name="c", subcore_axis_name="s")

    o_stripe = out_rows // 16
    seg_stripe = seg_rows // 16 if with_degs else 0
    g_stripe = g_rows // 16 if with_degs else 0

    def body(table_hbm, gidx2_hbm, sidx2_hbm, graw2_hbm, z2d_hbm, z1d_hbm,
             ones_hbm, *refs):
        refs = list(refs)
        if with_degs:
            acc_out, segdeg_out, gdeg_out = refs[:3]
            del refs[:3]
        else:
            acc_out = refs.pop(0)
        acc_sh = refs.pop(0)
        if table_spmem:
            table_sh = refs.pop(0)
        if with_degs:
            segdeg_sh = refs.pop(0)
            gdeg_sh = refs.pop(0)
        g0_v, g1_v, s0_v, s1_v = refs[:4]
        del refs[:4]
        if with_degs:
            gr0_v, gr1_v = refs[:2]
            del refs[:2]
            grbuf = (gr0_v, gr1_v)
        r0_v, r1_v, stage_v = refs[:3]
        del refs[:3]
        if with_degs:
            stage1_v, ones_v, de_pub = refs[:3]
            del refs[:3]
        gsem0, gsem1, ssem0, ssem1 = refs
        gbuf = (g0_v, g1_v)
        sbuf = (s0_v, s1_v)
        rbuf = (r0_v, r1_v)
        gsem = (gsem0, gsem1)
        ssem = (ssem0, ssem1)
        c = lax.axis_index("c")
        s = lax.axis_index("s")
        pltpu.sync_copy(z2d_hbm.at[pl.ds(0, _ZR)], stage_v)
        for k in range(o_stripe // _ZR):
            pltpu.sync_copy(stage_v,
                            acc_sh.at[pl.ds(s * o_stripe + k * _ZR, _ZR)])
        if with_degs:
            pltpu.sync_copy(z1d_hbm.at[pl.ds(0, max(seg_stripe, g_stripe))],
                            stage1_v)
            pltpu.sync_copy(stage1_v.at[pl.ds(0, seg_stripe)],
                            segdeg_sh.at[pl.ds(s * seg_stripe, seg_stripe)])
            pltpu.sync_copy(stage1_v.at[pl.ds(0, g_stripe)],
                            gdeg_sh.at[pl.ds(s * g_stripe, g_stripe)])
            pltpu.sync_copy(ones_hbm, ones_v)
        if table_spmem:
            t_stripe = table_rows // 16
            for k in range(t_stripe // _ZR):
                pltpu.sync_copy(
                    table_hbm.at[c, pl.ds(s * t_stripe + k * _ZR, _ZR)],
                    stage_v)
                pltpu.sync_copy(
                    stage_v,
                    table_sh.at[pl.ds(s * t_stripe + k * _ZR, _ZR)])
        plsc.subcore_barrier()
        gsrc = table_sh if table_spmem else table_hbm.at[c]

        def load_and_fire(g, b):
            row0 = s * (ppt // _IB) + g * _K
            pltpu.sync_copy(gidx2_hbm.at[pl.ds(row0, _K)], gbuf[b])
            pltpu.sync_copy(sidx2_hbm.at[pl.ds(row0, _K)], sbuf[b])
            if with_degs:
                pltpu.sync_copy(graw2_hbm.at[pl.ds(row0, _K)], grbuf[b])
            for j in range(_K):
                pltpu.async_copy(gsrc.at[gbuf[b].at[j]],
                                 rbuf[b].at[pl.ds(j * _IB, _IB)], gsem[b])

        def drain_gathers(b):
            for j in range(_K):
                pltpu.make_async_copy(
                    gsrc.at[gbuf[b].at[j]],
                    rbuf[b].at[pl.ds(j * _IB, _IB)], gsem[b]).wait()

        def scatter(b):
            pend = []
            for j in range(_K):
                pend.append(pltpu.async_copy(
                    rbuf[b].at[pl.ds(j * _IB, _IB)],
                    acc_sh.at[sbuf[b].at[j]], ssem[b], add=True))
                if with_degs:
                    pend.append(pltpu.async_copy(
                        ones_v, segdeg_sh.at[sbuf[b].at[j]], ssem[b],
                        add=True))
                    pend.append(pltpu.async_copy(
                        ones_v, gdeg_sh.at[grbuf[b].at[j]], ssem[b],
                        add=True))
            for d in pend:
                d.wait()

        load_and_fire(0, 0)

        def pair(gg, carry):
            for b in range(2):
                g = 2 * gg + b
                gnext = jnp.minimum(g + 1, chunks - 1)
                load_and_fire(gnext, 1 - b)
                drain_gathers(b)
                scatter(b)
            return carry

        lax.fori_loop(0, chunks // 2, pair, 0)
        drain_gathers(0)
        plsc.subcore_barrier()
        if with_degs:
            pltpu.sync_copy(
                segdeg_sh.at[pl.ds(s * seg_stripe, seg_stripe)], de_pub)
        for k in range(o_stripe // _ZR):
            pltpu.sync_copy(
                acc_sh.at[pl.ds(s * o_stripe + k * _ZR, _ZR)], stage_v)
            if with_degs:
                def _nrow(r, carry, k=k):
                    d = de_pub[k * _ZR + r]
                    inv = 1.0 / jnp.maximum(d, 1.0)
                    for q in range(fh // 16):
                        stage_v[r, pl.ds(q * 16, 16)] = (
                            stage_v[r, pl.ds(q * 16, 16)] * inv)
                    return carry
                lax.fori_loop(0, _ZR, _nrow, 0)
            pltpu.sync_copy(
                stage_v,
                acc_out.at[c, pl.ds(s * o_stripe + k * _ZR, _ZR)])
        if with_degs:
            pltpu.sync_copy(
                segdeg_sh.at[pl.ds(s * seg_stripe, seg_stripe)],
                stage1_v.at[pl.ds(0, seg_stripe)])
            pltpu.sync_copy(
                stage1_v.at[pl.ds(0, seg_stripe)],
                segdeg_out.at[c, pl.ds(s * seg_stripe, seg_stripe)])
            pltpu.sync_copy(gdeg_sh.at[pl.ds(s * g_stripe, g_stripe)],
                            stage1_v.at[pl.ds(0, g_stripe)])
            pltpu.sync_copy(
                stage1_v.at[pl.ds(0, g_stripe)],
                gdeg_out.at[c, pl.ds(s * g_stripe, g_stripe)])

    out_type = [jax.ShapeDtypeStruct((nc, out_rows, fh), jnp.float32)]
    scratch = [pltpu.VMEM_SHARED((out_rows, fh), jnp.float32)]
    if table_spmem:
        scratch += [pltpu.VMEM_SHARED((table_rows, fh), jnp.float32)]
    if with_degs:
        out_type += [
            jax.ShapeDtypeStruct((nc, seg_rows, 16), jnp.float32),
            jax.ShapeDtypeStruct((nc, g_rows, 16), jnp.float32)]
        scratch += [pltpu.VMEM_SHARED((seg_rows, 16), jnp.float32),
                    pltpu.VMEM_SHARED((g_rows, 16), jnp.float32)]
    scratch += [pltpu.VMEM((_K, _IB), jnp.int32),
                pltpu.VMEM((_K, _IB), jnp.int32),
                pltpu.VMEM((_K, _IB), jnp.int32),
                pltpu.VMEM((_K, _IB), jnp.int32)]
    if with_degs:
        scratch += [pltpu.VMEM((_K, _IB), jnp.int32),
                    pltpu.VMEM((_K, _IB), jnp.int32)]
    scratch += [pltpu.VMEM((_K * _IB, fh), jnp.float32),
                pltpu.VMEM((_K * _IB, fh), jnp.float32),
                pltpu.VMEM((_ZR, fh), jnp.float32)]
    if with_degs:
        scratch += [
            pltpu.VMEM((max(seg_rows, g_rows) // 16, 16), jnp.float32),
            pltpu.VMEM((_IB, 16), jnp.float32),
            pltpu.VMEM((seg_rows // 16, 16), jnp.float32)]
    scratch += [pltpu.SemaphoreType.DMA, pltpu.SemaphoreType.DMA,
                pltpu.SemaphoreType.DMA, pltpu.SemaphoreType.DMA]

    return pl.kernel(body, mesh=mesh, out_type=out_type,
                     scratch_types=scratch,
                     compiler_params=pltpu.CompilerParams(
                         use_tc_tiling_on_sc=False))


def _finish_kernel(op0_ref, op1_ref, dn_ref, w_ref, b_ref, out_ref):
    inv = 1.0 / jnp.maximum(dn_ref[...], 1.0)
    x = jnp.concatenate([op0_ref[...], op1_ref[...]], axis=1) * inv
    out_ref[...] = jnp.dot(x, w_ref[...],
                           preferred_element_type=jnp.float32) + b_ref[...]


def kernel(node_features, hyperedge_index, W, b):
    n, feat = node_features.shape
    e = hyperedge_index.shape[1]
    m = NUM_HYPEREDGES

    info = plsc.get_sparse_core_info()
    nc, ns = info.num_cores, info.num_subcores
    fh = feat // nc

    n_pad = _pad_len(n + 1, 16 * _ZR)
    m_pad = _pad_len(m + 1, 16 * _ZR)
    e_pad = _pad_len(e, 2 * ns * _K * _IB)

    node_idx = hyperedge_index[0]
    hedge_idx = hyperedge_index[1]
    if e_pad != e:
        node_idx = jnp.pad(node_idx, (0, e_pad - e), constant_values=n)
        hedge_idx = jnp.pad(hedge_idx, (0, e_pad - e), constant_values=m)
    x_pad = jnp.pad(node_features, ((0, n_pad - n), (0, 0)))
    x_split = x_pad.reshape(n_pad, nc, fh).transpose(1, 0, 2)

    nidx_2d = node_idx.reshape(-1, _IB)
    hidx_2d = hedge_idx.reshape(-1, _IB)

    z2d = jnp.zeros((_ZR, fh), jnp.float32)
    z1d = jnp.zeros((max(n_pad, m_pad) // 16 + 8, 16), jnp.float32)
    ones_c = jnp.ones((_IB, 16), jnp.float32)

    seg1 = _make_sc_segsum(e_pad, m_pad, nc, ns, fh,
                           seg_rows=m_pad, g_rows=n_pad, with_degs=True)
    hp, _, dn3 = seg1(x_split, nidx_2d, hidx_2d, nidx_2d, z2d, z1d,
                      ones_c)
    dn = dn3[0, :, 0]
    hf_split = hp

    seg2 = _make_sc_segsum(e_pad, n_pad, nc, ns, fh)
    op = seg2(hf_split, hidx_2d, nidx_2d, nidx_2d[:1], z2d, z1d, ones_c)
    if isinstance(op, (list, tuple)):
        op = op[0]

    bn = 400 if n % 400 == 0 else n
    out = pl.pallas_call(
        _finish_kernel,
        grid=(n // bn,),
        in_specs=[
            pl.BlockSpec((bn, fh), lambda i: (i, 0)),
            pl.BlockSpec((bn, fh), lambda i: (i, 0)),
            pl.BlockSpec((bn, 1), lambda i: (i, 0)),
            pl.BlockSpec((feat, feat), lambda i: (0, 0)),
            pl.BlockSpec((1, feat), lambda i: (0, 0)),
        ],
        out_specs=pl.BlockSpec((bn, feat), lambda i: (i, 0)),
        out_shape=jax.ShapeDtypeStruct((n, feat), jnp.float32),
    )(op[0, :n], op[1, :n], dn[:n, None], W, b[None, :])
    return out


# speedup vs baseline: 5.4407x; 1.0007x over previous
import jax
import jax.numpy as jnp
from jax import lax
from jax.experimental import pallas as pl
from jax.experimental.pallas import tpu as pltpu
from jax.experimental.pallas import tpu_sc as plsc

NUM_HYPEREDGES = 2500

_IB = 128
_K = 5
_ZR = 80


def _pad_len(n, mult):
    return ((n + mult - 1) // mult) * mult


def _make_sc_segsum(e_pad, out_rows, nc, ns, fh,
                    seg_rows=0, g_rows=0, with_degs=False,
                    table_rows=0):
    table_spmem = table_rows > 0
    blk = _K * _IB
    ppt = e_pad // ns
    chunks = ppt // blk

    mesh = plsc.VectorSubcoreMesh(core_axis_name="c", subcore_axis_name="s")

    o_stripe = out_rows // 16
    seg_stripe = seg_rows // 16 if with_degs else 0
    g_stripe = g_rows // 16 if with_degs else 0

    def body(table_hbm, gidx2_hbm, sidx2_hbm, graw2_hbm, z2d_hbm, z1d_hbm,
             ones_hbm, *refs):
        refs = list(refs)
        if with_degs:
            acc_out, segdeg_out, gdeg_out = refs[:3]
            del refs[:3]
        else:
            acc_out = refs.pop(0)
        acc_sh = refs.pop(0)
        if table_spmem:
            table_sh = refs.pop(0)
        if with_degs:
            segdeg_sh = refs.pop(0)
            gdeg_sh = refs.pop(0)
        g0_v, g1_v, s0_v, s1_v = refs[:4]
        del refs[:4]
        if with_degs:
            gr0_v, gr1_v = refs[:2]
            del refs[:2]
            grbuf = (gr0_v, gr1_v)
        r0_v, r1_v, stage_v = refs[:3]
        del refs[:3]
        if with_degs:
            stage1_v, ones_v, de_pub = refs[:3]
            del refs[:3]
        gsem0, gsem1, ssem0, ssem1 = refs
        gbuf = (g0_v, g1_v)
        sbuf = (s0_v, s1_v)
        rbuf = (r0_v, r1_v)
        gsem = (gsem0, gsem1)
        ssem = (ssem0, ssem1)
        c = lax.axis_index("c")
        s = lax.axis_index("s")
        pltpu.sync_copy(z2d_hbm.at[pl.ds(0, _ZR)], stage_v)
        for k in range(o_stripe // _ZR):
            pltpu.sync_copy(stage_v,
                            acc_sh.at[pl.ds(s * o_stripe + k * _ZR, _ZR)])
        if with_degs:
            pltpu.sync_copy(z1d_hbm.at[pl.ds(0, max(seg_stripe, g_stripe))],
                            stage1_v)
            pltpu.sync_copy(stage1_v.at[pl.ds(0, seg_stripe)],
                            segdeg_sh.at[pl.ds(s * seg_stripe, seg_stripe)])
            pltpu.sync_copy(stage1_v.at[pl.ds(0, g_stripe)],
                            gdeg_sh.at[pl.ds(s * g_stripe, g_stripe)])
            pltpu.sync_copy(ones_hbm, ones_v)
        if table_spmem:
            t_stripe = table_rows // 16
            for k in range(t_stripe // _ZR):
                pltpu.sync_copy(
                    table_hbm.at[c, pl.ds(s * t_stripe + k * _ZR, _ZR)],
                    stage_v)
                pltpu.sync_copy(
                    stage_v,
                    table_sh.at[pl.ds(s * t_stripe + k * _ZR, _ZR)])
        plsc.subcore_barrier()
        gsrc = table_sh if table_spmem else table_hbm.at[c]

        def load_and_fire(g, b):
            row0 = s * (ppt // _IB) + g * _K
            pltpu.sync_copy(gidx2_hbm.at[pl.ds(row0, _K)], gbuf[b])
            pltpu.sync_copy(sidx2_hbm.at[pl.ds(row0, _K)], sbuf[b])
            if with_degs:
                pltpu.sync_copy(graw2_hbm.at[pl.ds(row0, _K)], grbuf[b])
            for j in range(_K):
                pltpu.async_copy(gsrc.at[gbuf[b].at[j]],
                                 rbuf[b].at[pl.ds(j * _IB, _IB)], gsem[b])

        def drain_gathers(b):
            for j in range(_K):
                pltpu.make_async_copy(
                    gsrc.at[gbuf[b].at[j]],
                    rbuf[b].at[pl.ds(j * _IB, _IB)], gsem[b]).wait()

        def scatter(b):
            pend = []
            for j in range(_K):
                pend.append(pltpu.async_copy(
                    rbuf[b].at[pl.ds(j * _IB, _IB)],
                    acc_sh.at[sbuf[b].at[j]], ssem[b], add=True))
                if with_degs:
                    pend.append(pltpu.async_copy(
                        ones_v, segdeg_sh.at[sbuf[b].at[j]], ssem[b],
                        add=True))
                    pend.append(pltpu.async_copy(
                        ones_v, gdeg_sh.at[grbuf[b].at[j]], ssem[b],
                        add=True))
            for d in pend:
                d.wait()

        load_and_fire(0, 0)

        def pair(gg, carry):
            for b in range(2):
                g = 2 * gg + b
                gnext = jnp.minimum(g + 1, chunks - 1)
                load_and_fire(gnext, 1 - b)
                drain_gathers(b)
                scatter(b)
            return carry

        lax.fori_loop(0, chunks // 2, pair, 0)
        drain_gathers(0)
        plsc.subcore_barrier()
        if with_degs:
            pltpu.sync_copy(
                segdeg_sh.at[pl.ds(s * seg_stripe, seg_stripe)], de_pub)
        for k in range(o_stripe // _ZR):
            pltpu.sync_copy(
                acc_sh.at[pl.ds(s * o_stripe + k * _ZR, _ZR)], stage_v)
            if with_degs:
                def _nrow(r, carry, k=k):
                    d = de_pub[k * _ZR + r]
                    inv = 1.0 / jnp.maximum(d, 1.0)
                    for q in range(fh // 16):
                        stage_v[r, pl.ds(q * 16, 16)] = (
                            stage_v[r, pl.ds(q * 16, 16)] * inv)
                    return carry
                lax.fori_loop(0, _ZR, _nrow, 0)
            pltpu.sync_copy(
                stage_v,
                acc_out.at[c, pl.ds(s * o_stripe + k * _ZR, _ZR)])
        if with_degs:
            pltpu.sync_copy(
                segdeg_sh.at[pl.ds(s * seg_stripe, seg_stripe)],
                stage1_v.at[pl.ds(0, seg_stripe)])
            pltpu.sync_copy(
                stage1_v.at[pl.ds(0, seg_stripe)],
                segdeg_out.at[c, pl.ds(s * seg_stripe, seg_stripe)])
            pltpu.sync_copy(gdeg_sh.at[pl.ds(s * g_stripe, g_stripe)],
                            stage1_v.at[pl.ds(0, g_stripe)])
            pltpu.sync_copy(
                stage1_v.at[pl.ds(0, g_stripe)],
                gdeg_out.at[c, pl.ds(s * g_stripe, g_stripe)])

    out_type = [jax.ShapeDtypeStruct((nc, out_rows, fh), jnp.float32)]
    scratch = [pltpu.VMEM_SHARED((out_rows, fh), jnp.float32)]
    if table_spmem:
        scratch += [pltpu.VMEM_SHARED((table_rows, fh), jnp.float32)]
    if with_degs:
        out_type += [
            jax.ShapeDtypeStruct((nc, seg_rows, 16), jnp.float32),
            jax.ShapeDtypeStruct((nc, g_rows, 16), jnp.float32)]
        scratch += [pltpu.VMEM_SHARED((seg_rows, 16), jnp.float32),
                    pltpu.VMEM_SHARED((g_rows, 16), jnp.float32)]
    scratch += [pltpu.VMEM((_K, _IB), jnp.int32),
                pltpu.VMEM((_K, _IB), jnp.int32),
                pltpu.VMEM((_K, _IB), jnp.int32),
                pltpu.VMEM((_K, _IB), jnp.int32)]
    if with_degs:
        scratch += [pltpu.VMEM((_K, _IB), jnp.int32),
                    pltpu.VMEM((_K, _IB), jnp.int32)]
    scratch += [pltpu.VMEM((_K * _IB, fh), jnp.float32),
                pltpu.VMEM((_K * _IB, fh), jnp.float32),
                pltpu.VMEM((_ZR, fh), jnp.float32)]
    if with_degs:
        scratch += [
            pltpu.VMEM((max(seg_rows, g_rows) // 16, 16), jnp.float32),
            pltpu.VMEM((_IB, 16), jnp.float32),
            pltpu.VMEM((seg_rows // 16, 16), jnp.float32)]
    scratch += [pltpu.SemaphoreType.DMA, pltpu.SemaphoreType.DMA,
                pltpu.SemaphoreType.DMA, pltpu.SemaphoreType.DMA]

    return pl.kernel(body, mesh=mesh, out_type=out_type,
                     scratch_types=scratch,
                     compiler_params=pltpu.CompilerParams(
                         use_tc_tiling_on_sc=False))


def _finish_kernel(op0_ref, op1_ref, dn_ref, w_ref, b_ref, out_ref):
    inv = 1.0 / jnp.maximum(dn_ref[...], 1.0)
    x = jnp.concatenate([op0_ref[...], op1_ref[...]], axis=1) * inv
    out_ref[...] = jnp.dot(x, w_ref[...],
                           preferred_element_type=jnp.float32) + b_ref[...]


def kernel(node_features, hyperedge_index, W, b):
    n, feat = node_features.shape
    e = hyperedge_index.shape[1]
    m = NUM_HYPEREDGES

    info = plsc.get_sparse_core_info()
    nc, ns = info.num_cores, info.num_subcores
    fh = feat // nc

    n_pad = _pad_len(n + 1, 16 * _ZR)
    m_pad = _pad_len(m + 1, 16 * _ZR)
    e_pad = _pad_len(e, 2 * ns * _K * _IB)

    node_idx = hyperedge_index[0]
    hedge_idx = hyperedge_index[1]
    if e_pad != e:
        node_idx = jnp.pad(node_idx, (0, e_pad - e), constant_values=n)
        hedge_idx = jnp.pad(hedge_idx, (0, e_pad - e), constant_values=m)
    x_pad = jnp.pad(node_features, ((0, n_pad - n), (0, 0)))
    x_split = x_pad.reshape(n_pad, nc, fh).transpose(1, 0, 2)

    nidx_2d = node_idx.reshape(-1, _IB)
    hidx_2d = hedge_idx.reshape(-1, _IB)

    z2d = jnp.zeros((_ZR, fh), jnp.float32)
    z1d = jnp.zeros((max(n_pad, m_pad) // 16 + 8, 16), jnp.float32)
    ones_c = jnp.ones((_IB, 16), jnp.float32)

    seg1 = _make_sc_segsum(e_pad, m_pad, nc, ns, fh,
                           seg_rows=m_pad, g_rows=n_pad, with_degs=True)
    hp, _, dn3 = seg1(x_split, nidx_2d, hidx_2d, nidx_2d, z2d, z1d,
                      ones_c)
    dn = dn3[0, :, 0]
    hf_split = hp

    seg2 = _make_sc_segsum(e_pad, n_pad, nc, ns, fh)
    op = seg2(hf_split, hidx_2d, nidx_2d, nidx_2d[:1], z2d, z1d, ones_c)
    if isinstance(op, (list, tuple)):
        op = op[0]

    bn = 400 if n % 400 == 0 else n
    out = pl.pallas_call(
        _finish_kernel,
        grid=(n // bn,),
        in_specs=[
            pl.BlockSpec((bn, fh), lambda i: (i, 0)),
            pl.BlockSpec((bn, fh), lambda i: (i, 0)),
            pl.BlockSpec((bn, 1), lambda i: (i, 0)),
            pl.BlockSpec((feat, feat), lambda i: (0, 0)),
            pl.BlockSpec((1, feat), lambda i: (0, 0)),
        ],
        out_specs=pl.BlockSpec((bn, feat), lambda i: (i, 0)),
        out_shape=jax.ShapeDtypeStruct((n, feat), jnp.float32),
    )(op[0, :n], op[1, :n], dn[:n, None], W, b[None, :])
    return out
